# Initial kernel scaffold; baseline (speedup 1.0000x reference)
import jax
import jax.numpy as jnp
from jax.experimental import pallas as pl


def kernel(x, pos, edge_index, edge_attr, batch_indices, W_edge, W_enc, W_lin, b_lin):
    raise NotImplementedError("write your pallas kernel here")


# speedup vs baseline: 3.0448x; 3.0448x over previous
import functools

import jax
import jax.numpy as jnp
from jax import lax
from jax.experimental import pallas as pl
from jax.experimental.pallas import tpu as pltpu
from jax.experimental.pallas import tpu_sc as plsc


def _gate_body(ea_ref, we_ref, out_ref):
    g = jnp.dot(ea_ref[...], we_ref[...], preferred_element_type=jnp.float32)
    out_ref[...] = 1.0 / (1.0 + jnp.exp(-g))


def _gate(edge_attr, W_edge, block_e=2000):
    E, DE = edge_attr.shape
    D = W_edge.shape[1]
    return pl.pallas_call(
        _gate_body,
        grid=(E // block_e,),
        in_specs=[
            pl.BlockSpec((block_e, DE), lambda i: (i, 0)),
            pl.BlockSpec((DE, D), lambda i: (0, 0)),
        ],
        out_specs=pl.BlockSpec((block_e, D), lambda i: (i, 0)),
        out_shape=jax.ShapeDtypeStruct((E, D), jnp.float32),
    )(edge_attr, W_edge)


_NC = 2
_NS = 16
_CHUNK = 128


def _sc_edge_aggregate(x, src, dst, gate):
    N, D = x.shape
    E = src.shape[0]
    NW = _NC * _NS
    per_worker = E // NW
    n_full = per_worker // _CHUNK
    tail = per_worker - n_full * _CHUNK
    rows_per_tile = N // _NS
    zchunk = rows_per_tile // 5
    n_zero = 5

    z128 = jnp.zeros((_CHUNK, D), jnp.float32)
    zN = jnp.zeros((N,), jnp.float32)

    mesh = plsc.VectorSubcoreMesh(core_axis_name="c", subcore_axis_name="s")

    @functools.partial(
        pl.kernel,
        mesh=mesh,
        out_type=[
            jax.ShapeDtypeStruct((_NC, N, D), jnp.float32),
            jax.ShapeDtypeStruct((NW, N), jnp.float32),
        ],
        compiler_params=pltpu.CompilerParams(use_tc_tiling_on_sc=False,
                                             needs_layout_passes=False),
        scratch_types=[
            pltpu.VMEM((_CHUNK,), jnp.int32),
            pltpu.VMEM((_CHUNK,), jnp.int32),
            pltpu.VMEM((16,), jnp.int32),
            pltpu.VMEM((16,), jnp.int32),
            pltpu.VMEM((_CHUNK, D), jnp.float32),
            pltpu.VMEM((_CHUNK, D), jnp.float32),
            pltpu.VMEM((N,), jnp.float32),
            pltpu.VMEM_SHARED((N, D), jnp.float32),
            pltpu.SemaphoreType.DMA,
        ],
    )
    def sc_kernel(x_hbm, src_hbm, dst_hbm, gate_hbm, z128_hbm, zN_hbm,
                  agg_out, deg_out, src_v, dst_v, tsrc_v, tdst_v,
                  xrows_v, gate_v, deg_v, agg_sh, sem):
        cid = lax.axis_index("c")
        sid = lax.axis_index("s")

        pltpu.sync_copy(z128_hbm, xrows_v)
        pltpu.sync_copy(zN_hbm, deg_v)
        for q in range(n_zero):
            row = sid * rows_per_tile + q * zchunk
            pltpu.sync_copy(xrows_v.at[pl.ds(0, zchunk)],
                            agg_sh.at[pl.ds(row, zchunk), :])
        plsc.subcore_barrier()

        wid = cid * _NS + sid
        base = wid * per_worker
        ones16 = jnp.full((16,), 1.0, jnp.float32)

        def do_chunk(off, n, si_v, di_v):
            pltpu.sync_copy(src_hbm.at[pl.ds(off, n)], si_v)
            pltpu.sync_copy(dst_hbm.at[pl.ds(off, n)], di_v)
            pltpu.async_copy(x_hbm.at[si_v], xrows_v.at[pl.ds(0, n)],
                             sem).wait()
            pltpu.sync_copy(gate_hbm.at[pl.ds(off, n), :],
                            gate_v.at[pl.ds(0, n)])

            def mul_row(r, carry):
                for k in range(D // 16):
                    s = pl.ds(k * 16, 16)
                    xrows_v[r, s] = xrows_v[r, s] * gate_v[r, s]
                return carry

            lax.fori_loop(0, n, mul_row, 0)
            for j in range(n // 16):
                idx16 = di_v[pl.ds(j * 16, 16)]
                plsc.addupdate_scatter(deg_v, [idx16], ones16)
            pltpu.sync_copy(xrows_v.at[pl.ds(0, n)], agg_sh.at[di_v],
                            add=True)

        def step(j, carry):
            do_chunk(base + j * _CHUNK, _CHUNK, src_v, dst_v)
            return carry

        lax.fori_loop(0, n_full, step, 0)
        if tail:
            do_chunk(base + n_full * _CHUNK, tail, tsrc_v, tdst_v)

        plsc.subcore_barrier()
        pltpu.sync_copy(deg_v, deg_out.at[wid])
        for q in range(n_zero):
            row = sid * rows_per_tile + q * zchunk
            pltpu.sync_copy(agg_sh.at[pl.ds(row, zchunk), :],
                            xrows_v.at[pl.ds(0, zchunk)])
            pltpu.sync_copy(xrows_v.at[pl.ds(0, zchunk)],
                            agg_out.at[cid, pl.ds(row, zchunk), :])

    return sc_kernel(x, src, dst, gate, z128, zN)


def _head(x, agg0, agg1, deg_parts, batch_indices, W_enc, W_lin, b_lin,
          n_groups, block_n=1000):
    N, D = x.shape
    NW = deg_parts.shape[1]
    nb = N // block_n
    bi3 = batch_indices.reshape(nb, 1, block_n)
    bl2 = b_lin.reshape(1, 1)

    def body(x_ref, a0_ref, a1_ref, d_ref, bi_ref, we_ref, wl_ref,
             bl_ref, out_ref, pooled_acc, cnt_acc):
        i = pl.program_id(0)

        @pl.when(i == 0)
        def _():
            pooled_acc[...] = jnp.zeros_like(pooled_acc)
            cnt_acc[...] = jnp.zeros_like(cnt_acc)

        agg = a0_ref[...] + a1_ref[...]
        deg = jnp.sum(d_ref[...], axis=1, keepdims=True)
        agg = agg / jnp.maximum(deg, 1.0)
        h = jnp.maximum(
            jnp.dot(x_ref[...] + agg, we_ref[...],
                    preferred_element_type=jnp.float32), 0.0)
        b = bi_ref[0, 0, :]
        onehot = (b[:, None] == lax.broadcasted_iota(
            jnp.int32, (block_n, n_groups), 1)).astype(jnp.float32)
        pooled_acc[...] += lax.dot_general(
            onehot, h, (((0,), (0,)), ((), ())),
            preferred_element_type=jnp.float32)
        cnt = jnp.sum(onehot, axis=0)
        cnt_acc[...] += cnt[:, None]

        pooled = pooled_acc[...] / jnp.maximum(cnt_acc[...], 1.0)
        out_ref[...] = jnp.dot(pooled, wl_ref[...],
                               preferred_element_type=jnp.float32) + bl_ref[0, 0]

    return pl.pallas_call(
        body,
        grid=(nb,),
        in_specs=[
            pl.BlockSpec((block_n, D), lambda i: (i, 0)),
            pl.BlockSpec((block_n, D), lambda i: (i, 0)),
            pl.BlockSpec((block_n, D), lambda i: (i, 0)),
            pl.BlockSpec((block_n, NW), lambda i: (i, 0)),
            pl.BlockSpec((1, 1, block_n), lambda i: (i, 0, 0)),
            pl.BlockSpec((D, D), lambda i: (0, 0)),
            pl.BlockSpec((D, 1), lambda i: (0, 0)),
            pl.BlockSpec(memory_space=pltpu.SMEM),
        ],
        out_specs=pl.BlockSpec((n_groups, 1), lambda i: (0, 0)),
        out_shape=jax.ShapeDtypeStruct((n_groups, 1), jnp.float32),
        scratch_shapes=[
            pltpu.VMEM((n_groups, D), jnp.float32),
            pltpu.VMEM((n_groups, D), jnp.float32),
        ],
    )(x, agg0, agg1, deg_parts, bi3, W_enc, W_lin, bl2)


def kernel(x, pos, edge_index, edge_attr, batch_indices, W_edge, W_enc,
           W_lin, b_lin):
    del pos
    N, D = x.shape
    G = 64

    gate = _gate(edge_attr, W_edge)
    src = edge_index[0]
    dst = edge_index[1]
    agg_parts, deg_parts = _sc_edge_aggregate(x, src, dst, gate)
    out = _head(x, agg_parts[0], agg_parts[1], deg_parts.T,
                batch_indices, W_enc, W_lin, b_lin, G)
    return out
